# Initial kernel scaffold; baseline (speedup 1.0000x reference)
import jax
import jax.numpy as jnp
from jax.experimental import pallas as pl


def kernel(positions, pe):
    raise NotImplementedError("write your pallas kernel here")


# speedup vs baseline: 1.5767x; 1.5767x over previous
import functools

import jax
import jax.numpy as jnp
from jax import lax
from jax.experimental import pallas as pl
from jax.experimental.pallas import tpu as pltpu
from jax.experimental.pallas import tpu_sc as plsc

_EMB = 1024
_SEQ = 8192
_NUM_CORES = 2
_NUM_SUBCORES = 16
_NW = _NUM_CORES * _NUM_SUBCORES
_B_PER_W = _SEQ // _NW
_CHUNK = 32
_NCHUNK = _B_PER_W // _CHUNK
_NBUF = 3

_mesh = plsc.VectorSubcoreMesh(core_axis_name="c", subcore_axis_name="s")


@functools.partial(
    pl.kernel,
    mesh=_mesh,
    out_type=jax.ShapeDtypeStruct((_SEQ, _EMB), jnp.float32),
    scratch_types=[
        pltpu.VMEM((_B_PER_W,), jnp.int32),
        pltpu.VMEM((_NBUF, _CHUNK, _EMB), jnp.float32),
        pltpu.SemaphoreType.DMA((_NBUF,)),
        pltpu.SemaphoreType.DMA((_NBUF,)),
    ],
)
def _gather_rows(pe_hbm, pos_hbm, out_hbm, idx_v, bufs, gsems, wsems):
    wid = lax.axis_index("s") * _NUM_CORES + lax.axis_index("c")
    base = wid * _B_PER_W
    pltpu.sync_copy(pos_hbm.at[pl.ds(base, _B_PER_W)], idx_v)

    def start_gather(i):
        slot = i % _NBUF
        return pltpu.async_copy(
            pe_hbm.at[idx_v.at[pl.ds(i * _CHUNK, _CHUNK)]],
            bufs.at[slot],
            gsems.at[slot],
        )

    gathers = [None] * _NCHUNK
    writes = [None] * _NCHUNK
    for i in range(_NBUF):
        gathers[i] = start_gather(i)
    for i in range(_NCHUNK):
        slot = i % _NBUF
        gathers[i].wait()
        writes[i] = pltpu.async_copy(
            bufs.at[slot],
            out_hbm.at[pl.ds(base + i * _CHUNK, _CHUNK)],
            wsems.at[slot],
        )
        nxt = i + _NBUF
        if nxt < _NCHUNK:
            writes[i].wait()
            gathers[nxt] = start_gather(nxt)
    for i in range(_NCHUNK - _NBUF, _NCHUNK):
        writes[i].wait()


def kernel(positions, pe):
    return _gather_rows(pe, positions)


# speedup vs baseline: 1.5804x; 1.0024x over previous
import functools

import jax
import jax.numpy as jnp
from jax import lax
from jax.experimental import pallas as pl
from jax.experimental.pallas import tpu as pltpu
from jax.experimental.pallas import tpu_sc as plsc

_EMB = 1024
_SEQ = 8192
_NUM_CORES = 2
_NUM_SUBCORES = 16
_NW = _NUM_CORES * _NUM_SUBCORES
_B_PER_W = _SEQ // _NW
_CHUNK = 16
_NCHUNK = _B_PER_W // _CHUNK
_NBUF = 7

_mesh = plsc.VectorSubcoreMesh(core_axis_name="c", subcore_axis_name="s")


@functools.partial(
    pl.kernel,
    mesh=_mesh,
    out_type=jax.ShapeDtypeStruct((_SEQ, _EMB), jnp.float32),
    scratch_types=[
        pltpu.VMEM((_B_PER_W,), jnp.int32),
        pltpu.VMEM((_NBUF, _CHUNK, _EMB), jnp.float32),
        pltpu.SemaphoreType.DMA((_NBUF,)),
        pltpu.SemaphoreType.DMA((_NBUF,)),
    ],
)
def _gather_rows(pe_hbm, pos_hbm, out_hbm, idx_v, bufs, gsems, wsems):
    wid = lax.axis_index("s") * _NUM_CORES + lax.axis_index("c")
    base = wid * _B_PER_W
    pltpu.sync_copy(pos_hbm.at[pl.ds(base, _B_PER_W)], idx_v)

    def start_gather(i):
        slot = i % _NBUF
        return pltpu.async_copy(
            pe_hbm.at[idx_v.at[pl.ds(i * _CHUNK, _CHUNK)]],
            bufs.at[slot],
            gsems.at[slot],
        )

    gathers = [None] * _NCHUNK
    writes = [None] * _NCHUNK
    for i in range(_NBUF):
        gathers[i] = start_gather(i)
    for i in range(_NCHUNK):
        slot = i % _NBUF
        gathers[i].wait()
        writes[i] = pltpu.async_copy(
            bufs.at[slot],
            out_hbm.at[pl.ds(base + i * _CHUNK, _CHUNK)],
            wsems.at[slot],
        )
        nxt = i + _NBUF
        if nxt < _NCHUNK:
            writes[i].wait()
            gathers[nxt] = start_gather(nxt)
    for i in range(_NCHUNK - _NBUF, _NCHUNK):
        writes[i].wait()


def kernel(positions, pe):
    return _gather_rows(pe, positions)
